# baseline (speedup 1.0000x reference)
import jax
import jax.numpy as jnp
from jax.experimental import pallas as pl
from jax.experimental.pallas import tpu as pltpu

_N = 1024
_TAIL = 256
_NMASK = 768
_R = 8


def _threefry_bits(f):
    ks0 = jnp.uint32(0)
    ks1 = jnp.uint32(42)
    ks2 = jnp.uint32(0x1BD11BDA) ^ ks0 ^ ks1
    ks = (ks0, ks1, ks2)
    rot = ((13, 15, 26, 6), (17, 29, 16, 24))
    x0 = jnp.broadcast_to(ks0, f.shape)
    x1 = f + ks1
    for i in range(5):
        for r in rot[i % 2]:
            x0 = x0 + x1
            x1 = (x1 << r) | (x1 >> (32 - r))
            x1 = x0 ^ x1
        x0 = x0 + ks[(i + 1) % 3]
        x1 = x1 + ks[(i + 2) % 3] + jnp.uint32(i + 1)
    return x0 ^ x1


def _mask_body(out_ref, mant_ref):
    g = pl.program_id(0)
    row0 = g * _R

    lane = jax.lax.broadcasted_iota(jnp.int32, (_R, _N), 1)
    sub = jax.lax.broadcasted_iota(jnp.int32, (_R, _N), 0)
    f = ((row0 + sub) * _N + lane).astype(jnp.uint32)
    mant_ref[...] = (_threefry_bits(f) >> 9).astype(jnp.int32)

    tcol = jax.lax.broadcasted_iota(jnp.int32, (_TAIL, 1), 0)
    lane_t = jax.lax.broadcasted_iota(jnp.int32, (_TAIL, _N), 1)
    tie_lt = lane_t < (_NMASK + tcol)

    def row_step(r, carry):
        rowm = mant_ref[pl.ds(r, 1), :]
        ft = (((row0 + r) * _N + _NMASK) + tcol).astype(jnp.uint32)
        tailm = (_threefry_bits(ft) >> 9).astype(jnp.int32)
        cmp = (rowm < tailm) | ((rowm == tailm) & tie_lt)
        ranks = jnp.sum(cmp.astype(jnp.int32), axis=1, keepdims=True)
        scat = jnp.any(ranks == lane_t, axis=0, keepdims=True)
        out_ref[pl.ds(r, 1), :] = jnp.where(scat, 0, 1).astype(jnp.int32)
        return carry

    jax.lax.fori_loop(0, _R, row_step, 0)


def kernel(x):
    del x
    out = pl.pallas_call(
        _mask_body,
        grid=(_N // _R,),
        out_shape=jax.ShapeDtypeStruct((_N, _N), jnp.int32),
        out_specs=pl.BlockSpec((_R, _N), lambda g: (g, 0)),
        scratch_shapes=[pltpu.VMEM((_R, _N), jnp.int32)],
    )()
    return out.astype(bool)


# speedup vs baseline: 6.5688x; 6.5688x over previous
import functools

import jax
import jax.numpy as jnp
from jax import lax
from jax.experimental import pallas as pl
from jax.experimental.pallas import tpu as pltpu
from jax.experimental.pallas import tpu_sc as plsc

_N = 1024
_NMASK = 768
_TAIL = _N - _NMASK
_CAP = 9
_G = 16
_NGRP = _N // _G


def _threefry_bits(f):
    ks0 = jnp.uint32(0)
    ks1 = jnp.uint32(42)
    ks2 = jnp.uint32(0x1BD11BDA) ^ ks0 ^ ks1
    ks = (ks0, ks1, ks2)
    rot = ((13, 15, 26, 6), (17, 29, 16, 24))
    x0 = jnp.broadcast_to(ks0, f.shape)
    x1 = f + ks1
    for i in range(5):
        for r in rot[i % 2]:
            x0 = x0 + x1
            x1 = (x1 << r) | (x1 >> (32 - r))
            x1 = x0 ^ x1
        x0 = x0 + ks[(i + 1) % 3]
        x1 = x1 + ks[(i + 2) % 3] + jnp.uint32(i + 1)
    return x0 ^ x1


def _mant_body(out_ref):
    g = pl.program_id(0)
    a = lax.broadcasted_iota(jnp.int32, (1, 128, 128), 1)
    b = lax.broadcasted_iota(jnp.int32, (1, 128, 128), 2)
    j = a * 8 + (b >> 4)
    i = g * _G + (b & 15)
    f = (i * _N + j).astype(jnp.uint32)
    out_ref[...] = (_threefry_bits(f) >> 9).astype(jnp.int32)


_sc_mesh = plsc.VectorSubcoreMesh(core_axis_name="c", subcore_axis_name="s")


@functools.partial(
    pl.kernel,
    out_type=jax.ShapeDtypeStruct((_N, _N), jnp.int32),
    mesh=_sc_mesh,
    scratch_types=[
        pltpu.VMEM((128, 128), jnp.int32),
        pltpu.VMEM((_N * _G,), jnp.int32),
        pltpu.VMEM((_N * _G,), jnp.int32),
        pltpu.VMEM((_N * _G,), jnp.int32),
        pltpu.VMEM((_G, _N), jnp.int32),
    ],
    compiler_params=pltpu.CompilerParams(needs_layout_passes=False),
)
def _sc_mask(mant_hbm, out_hbm, mant_v, hist_v, cur_v, s_v, out_v):
    wid = lax.axis_index("s") * 2 + lax.axis_index("c")
    lanes = lax.iota(jnp.int32, 16)
    ones = jnp.ones((_G,), jnp.int32)
    zeros = jnp.zeros((_G,), jnp.int32)

    def _mant(j):
        return mant_v[j >> 3, pl.ds((j & 7) * _G, _G)]

    for g in range(2):
        grp = wid * 2 + g
        base = grp * _G
        pltpu.sync_copy(mant_hbm.at[grp], mant_v)

        def zero_body(b, _):
            hist_v[pl.ds(b * _G, _G)] = zeros
            return 0

        lax.fori_loop(0, _N, zero_body, 0, unroll=4)

        def hist_body(j, _):
            addr = ((mant_v[j >> 3, pl.ds((j & 7) * _G, _G)] >> 13) << 4) + lanes
            plsc.addupdate_scatter(hist_v, [addr], ones)
            return 0

        lax.fori_loop(0, _N, hist_body, 0, unroll=4)

        def pref_body(b, acc):
            c = hist_v[pl.ds(b * _G, _G)]
            cur_v[pl.ds(b * _G, _G)] = acc
            return acc + c

        lax.fori_loop(0, _N, pref_body, zeros, unroll=4)

        def place_body(j, _):
            m = mant_v[j >> 3, pl.ds((j & 7) * _G, _G)]
            addr = ((m >> 13) << 4) + lanes
            k2 = ((m & 0x1FFF) << 10) | j
            cur = plsc.load_gather(cur_v, [addr])
            plsc.store_scatter(s_v, [(cur << 4) + lanes], k2)
            plsc.addupdate_scatter(cur_v, [addr], ones)
            return 0

        lax.fori_loop(0, _N, place_body, 0, unroll=2)

        for r in range(_G):

            def oinit_body(c, _, r=r):
                out_v[r, pl.ds(c * _G, _G)] = ones
                return 0

            lax.fori_loop(0, _N // _G, oinit_body, 0, unroll=4)

        def tail_body(t, _):
            j = t + _NMASK
            m = mant_v[j >> 3, pl.ds((j & 7) * _G, _G)]
            addr = ((m >> 13) << 4) + lanes
            k2k = ((m & 0x1FFF) << 10) | j
            endc = plsc.load_gather(cur_v, [addr])
            cnt = plsc.load_gather(hist_v, [addr])
            start = endc - cnt
            start16 = (start << 4) + lanes
            fine = zeros
            for c in range(_CAP):
                msk = cnt > c
                occ = plsc.load_gather(s_v, [start16 + c * _G], mask=msk)
                fine = fine + jnp.where(msk & (occ < k2k), 1, 0)
            plsc.store_scatter(out_v, [lanes, start + fine], zeros)
            return 0

        lax.fori_loop(0, _TAIL, tail_body, 0)

        pltpu.sync_copy(out_v, out_hbm.at[pl.ds(base, _G), :])


def kernel(x):
    del x
    mant_g = pl.pallas_call(
        _mant_body,
        grid=(_NGRP,),
        out_shape=jax.ShapeDtypeStruct((_NGRP, 128, 128), jnp.int32),
        out_specs=pl.BlockSpec((1, 128, 128), lambda g: (g, 0, 0)),
    )()
    return _sc_mask(mant_g).astype(bool)


# speedup vs baseline: 9.3433x; 1.4224x over previous
import functools

import jax
import jax.numpy as jnp
from jax import lax
from jax.experimental import pallas as pl
from jax.experimental.pallas import tpu as pltpu
from jax.experimental.pallas import tpu_sc as plsc

_N = 1024
_NMASK = 768
_TAIL = _N - _NMASK
_CAP = 9
_G = 16
_NGRP = _N // _G


def _threefry_bits(f):
    ks0 = jnp.uint32(0)
    ks1 = jnp.uint32(42)
    ks2 = jnp.uint32(0x1BD11BDA) ^ ks0 ^ ks1
    ks = (ks0, ks1, ks2)
    rot = ((13, 15, 26, 6), (17, 29, 16, 24))
    x0 = jnp.broadcast_to(ks0, f.shape)
    x1 = f + ks1
    for i in range(5):
        for r in rot[i % 2]:
            x0 = x0 + x1
            x1 = (x1 << r) | (x1 >> (32 - r))
            x1 = x0 ^ x1
        x0 = x0 + ks[(i + 1) % 3]
        x1 = x1 + ks[(i + 2) % 3] + jnp.uint32(i + 1)
    return x0 ^ x1


_TCB = 4


def _mant_body(out_ref):
    g = pl.program_id(0)
    q = lax.broadcasted_iota(jnp.int32, (_TCB, 128, 128), 0)
    a = lax.broadcasted_iota(jnp.int32, (_TCB, 128, 128), 1)
    b = lax.broadcasted_iota(jnp.int32, (_TCB, 128, 128), 2)
    j = a * 8 + (b >> 4)
    i = (g * _TCB + q) * _G + (b & 15)
    f = (i * _N + j).astype(jnp.uint32)
    out_ref[...] = (_threefry_bits(f) >> 9).astype(jnp.int32)


_sc_mesh = plsc.VectorSubcoreMesh(core_axis_name="c", subcore_axis_name="s")


@functools.partial(
    pl.kernel,
    out_type=jax.ShapeDtypeStruct((_N, _N), jnp.int32),
    mesh=_sc_mesh,
    scratch_types=[
        pltpu.VMEM((128, 128), jnp.int32),
        pltpu.VMEM((_N * _G,), jnp.int32),
        pltpu.VMEM((_N * _G,), jnp.int32),
        pltpu.VMEM((_N * _G,), jnp.int32),
        pltpu.VMEM((_G, _N), jnp.int32),
    ],
    compiler_params=pltpu.CompilerParams(needs_layout_passes=False),
)
def _sc_mask(mant_hbm, out_hbm, mant_v, hist_v, cur_v, s_v, out_v):
    wid = lax.axis_index("s") * 2 + lax.axis_index("c")
    lanes = lax.iota(jnp.int32, 16)
    ones = jnp.ones((_G,), jnp.int32)
    zeros = jnp.zeros((_G,), jnp.int32)

    def _mant(j):
        return mant_v[j >> 3, pl.ds((j & 7) * _G, _G)]

    for g in range(2):
        grp = wid * 2 + g
        base = grp * _G
        pltpu.sync_copy(mant_hbm.at[grp], mant_v)

        @plsc.parallel_loop(0, _N, unroll=8)
        def _(b):
            hist_v[pl.ds(b * _G, _G)] = zeros

        @plsc.parallel_loop(0, _N, unroll=8)
        def _(j):
            addr = ((mant_v[j >> 3, pl.ds((j & 7) * _G, _G)] >> 13) << 4) + lanes
            plsc.addupdate_scatter(hist_v, [addr], ones)

        @plsc.parallel_loop(0, _N, unroll=4, carry=zeros)
        def _(b, acc):
            c = hist_v[pl.ds(b * _G, _G)]
            cur_v[pl.ds(b * _G, _G)] = acc
            return acc + c

        def place_body(j, _):
            m = mant_v[j >> 3, pl.ds((j & 7) * _G, _G)]
            addr = ((m >> 13) << 4) + lanes
            k2 = ((m & 0x1FFF) << 10) | j
            cur = plsc.load_gather(cur_v, [addr])
            plsc.store_scatter(s_v, [(cur << 4) + lanes], k2)
            plsc.addupdate_scatter(cur_v, [addr], ones)
            return 0

        lax.fori_loop(0, _N, place_body, 0, unroll=4)

        for r in range(_G):

            @plsc.parallel_loop(0, _N // _G, unroll=8)
            def _(c, r=r):
                out_v[r, pl.ds(c * _G, _G)] = ones

        @plsc.parallel_loop(0, _TAIL, unroll=2)
        def _(t):
            j = t + _NMASK
            m = mant_v[j >> 3, pl.ds((j & 7) * _G, _G)]
            addr = ((m >> 13) << 4) + lanes
            k2k = ((m & 0x1FFF) << 10) | j
            endc = plsc.load_gather(cur_v, [addr])
            cnt = plsc.load_gather(hist_v, [addr])
            start = endc - cnt
            start16 = (start << 4) + lanes
            fine = zeros
            for c in range(_CAP):
                msk = cnt > c
                occ = plsc.load_gather(s_v, [start16 + c * _G], mask=msk)
                fine = fine + jnp.where(msk & (occ < k2k), 1, 0)
            plsc.store_scatter(out_v, [lanes, start + fine], zeros)

        pltpu.sync_copy(out_v, out_hbm.at[pl.ds(base, _G), :])


def kernel(x):
    del x
    mant_g = pl.pallas_call(
        _mant_body,
        grid=(_NGRP // _TCB,),
        out_shape=jax.ShapeDtypeStruct((_NGRP, 128, 128), jnp.int32),
        out_specs=pl.BlockSpec((_TCB, 128, 128), lambda g: (g, 0, 0)),
    )()
    return _sc_mask(mant_g).astype(bool)


# speedup vs baseline: 9.7292x; 1.0413x over previous
import functools

import jax
import jax.numpy as jnp
from jax import lax
from jax.experimental import pallas as pl
from jax.experimental.pallas import tpu as pltpu
from jax.experimental.pallas import tpu_sc as plsc

_N = 1024
_NMASK = 768
_TAIL = _N - _NMASK
_CAP = 9
_G = 16
_NGRP = _N // _G


def _threefry_bits(f):
    ks0 = jnp.uint32(0)
    ks1 = jnp.uint32(42)
    ks2 = jnp.uint32(0x1BD11BDA) ^ ks0 ^ ks1
    ks = (ks0, ks1, ks2)
    rot = ((13, 15, 26, 6), (17, 29, 16, 24))
    x0 = jnp.broadcast_to(ks0, f.shape)
    x1 = f + ks1
    for i in range(5):
        for r in rot[i % 2]:
            x0 = x0 + x1
            x1 = (x1 << r) | (x1 >> (32 - r))
            x1 = x0 ^ x1
        x0 = x0 + ks[(i + 1) % 3]
        x1 = x1 + ks[(i + 2) % 3] + jnp.uint32(i + 1)
    return x0 ^ x1


_TCB = 4


def _mant_body(out_ref, *, grp0):
    g = pl.program_id(0)
    q = lax.broadcasted_iota(jnp.int32, (_TCB, 128, 128), 0)
    a = lax.broadcasted_iota(jnp.int32, (_TCB, 128, 128), 1)
    b = lax.broadcasted_iota(jnp.int32, (_TCB, 128, 128), 2)
    j = a * 8 + (b >> 4)
    i = (grp0 + g * _TCB + q) * _G + (b & 15)
    f = (i * _N + j).astype(jnp.uint32)
    out_ref[...] = (_threefry_bits(f) >> 9).astype(jnp.int32)


_sc_mesh = plsc.VectorSubcoreMesh(core_axis_name="c", subcore_axis_name="s")


@functools.partial(
    pl.kernel,
    out_type=jax.ShapeDtypeStruct((_N // 2, _N), jnp.int32),
    mesh=_sc_mesh,
    scratch_types=[
        pltpu.VMEM((128, 128), jnp.int32),
        pltpu.VMEM((_N * _G,), jnp.int32),
        pltpu.VMEM((_N * _G,), jnp.int32),
        pltpu.VMEM((_N * _G,), jnp.int32),
        pltpu.VMEM((_G, _N), jnp.int32),
    ],
    compiler_params=pltpu.CompilerParams(needs_layout_passes=False),
)
def _sc_mask(mant_hbm, out_hbm, mant_v, hist_v, cur_v, s_v, out_v):
    wid = lax.axis_index("s") * 2 + lax.axis_index("c")
    lanes = lax.iota(jnp.int32, 16)
    ones = jnp.ones((_G,), jnp.int32)
    zeros = jnp.zeros((_G,), jnp.int32)

    if True:
        grp = wid
        base = grp * _G
        pltpu.sync_copy(mant_hbm.at[grp], mant_v)

        @plsc.parallel_loop(0, _N, unroll=8)
        def _(b):
            hist_v[pl.ds(b * _G, _G)] = zeros

        @plsc.parallel_loop(0, _N, unroll=8)
        def _(j):
            addr = ((mant_v[j >> 3, pl.ds((j & 7) * _G, _G)] >> 13) << 4) + lanes
            plsc.addupdate_scatter(hist_v, [addr], ones)

        @plsc.parallel_loop(0, _N, unroll=4, carry=zeros)
        def _(b, acc):
            c = hist_v[pl.ds(b * _G, _G)]
            cur_v[pl.ds(b * _G, _G)] = acc
            return acc + c

        def place_body(j, _):
            m = mant_v[j >> 3, pl.ds((j & 7) * _G, _G)]
            addr = ((m >> 13) << 4) + lanes
            k2 = ((m & 0x1FFF) << 10) | j
            cur = plsc.load_gather(cur_v, [addr])
            plsc.store_scatter(s_v, [(cur << 4) + lanes], k2)
            plsc.addupdate_scatter(cur_v, [addr], ones)
            return 0

        lax.fori_loop(0, _N, place_body, 0, unroll=4)

        for r in range(_G):

            @plsc.parallel_loop(0, _N // _G, unroll=8)
            def _(c, r=r):
                out_v[r, pl.ds(c * _G, _G)] = ones

        @plsc.parallel_loop(0, _TAIL, unroll=2)
        def _(t):
            j = t + _NMASK
            m = mant_v[j >> 3, pl.ds((j & 7) * _G, _G)]
            addr = ((m >> 13) << 4) + lanes
            k2k = ((m & 0x1FFF) << 10) | j
            endc = plsc.load_gather(cur_v, [addr])
            cnt = plsc.load_gather(hist_v, [addr])
            start = endc - cnt
            start16 = (start << 4) + lanes
            fine = zeros
            for c in range(_CAP):
                msk = cnt > c
                occ = plsc.load_gather(s_v, [start16 + c * _G], mask=msk)
                fine = fine + jnp.where(msk & (occ < k2k), 1, 0)
            plsc.store_scatter(out_v, [lanes, start + fine], zeros)

        pltpu.sync_copy(out_v, out_hbm.at[pl.ds(base, _G), :])


def kernel(x):
    del x
    halves = []
    for h in range(2):
        mant_h = pl.pallas_call(
            functools.partial(_mant_body, grp0=h * (_NGRP // 2)),
            grid=(_NGRP // 2 // _TCB,),
            out_shape=jax.ShapeDtypeStruct((_NGRP // 2, 128, 128), jnp.int32),
            out_specs=pl.BlockSpec((_TCB, 128, 128), lambda g: (g, 0, 0)),
        )()
        halves.append(_sc_mask(mant_h))
    return jnp.concatenate(halves, axis=0).astype(bool)
